# baseline (speedup 1.0000x reference)
import dataclasses

import jax
import jax.numpy as jnp
from jax import lax
from jax.experimental import pallas as pl
from jax.experimental.pallas import tpu as pltpu
from jax.experimental.pallas import tpu_sc as plsc

ROWS_IN = 65536
COLS = 1024
ROWS_OUT = ROWS_IN - 1
COLS_OUT = COLS + 1

NUM_WORKERS = 32
L = 16
GR = 128
NCH = GR // L
PC = 128
NPANEL = COLS // PC
ROWS_PER_WORKER = 2048
GROUPS_FULL = ROWS_PER_WORKER // GR
ROWS_SC = 65408
TAIL = ROWS_OUT - ROWS_SC


def _sc_kernel(x_hbm, out_hbm, tot_hbm,
               xb0, xb1, ob0, ob1, tot_buf,
               sem_x0, sem_x1, sem_o0, sem_o1):
    c = lax.axis_index("c")
    s = lax.axis_index("s")
    wid = s * 2 + c
    base = wid * ROWS_PER_WORKER

    iota = jnp.arange(L, dtype=jnp.int32)
    rows_h = [iota + L * h for h in range(NCH)]
    zeros_f = jnp.zeros((L,), jnp.float32)

    ngroups = jnp.where(wid == NUM_WORKERS - 1,
                        GROUPS_FULL - 1, GROUPS_FULL)
    nunits = ngroups * NPANEL
    npairs = nunits // 2

    def unit_slices(u):
        g = u // NPANEL
        p = lax.rem(u, NPANEL)
        return g, p, base + g * GR, p * PC

    def issue_x(u, xb, sem):
        _, _, gr0, c0 = unit_slices(u)
        pltpu.async_copy(x_hbm.at[pl.ds(gr0, GR), pl.ds(c0, PC)], xb, sem)

    def sweep(u, xb, ob):
        g, p, gr0, c0 = unit_slices(u)
        keep = jnp.where(p == 0, 0.0, 1.0)
        accs = [tot_buf[pl.ds(g * GR + L * h, L)] * keep for h in range(NCH)]

        zvec = jnp.zeros((L,), jnp.int32)

        def unpack(carry):
            accs = list(carry[0:NCH])
            gidx = list(carry[NCH:2 * NCH])
            sidx = list(carry[2 * NCH:3 * NCH])
            return accs, gidx, sidx, carry[-1]

        def step_masked(carry):
            accs, gidx, sidx, jv = unpack(carry)
            m = (jv >= 0) & (jv < PC)
            for h in range(NCH):
                plsc.store_scatter(ob, [zvec, sidx[h]], accs[h], mask=m)
                v = plsc.load_gather(xb, [zvec, gidx[h]], mask=m)
                accs[h] = accs[h] + jnp.where(m, v, 0.0)
                gidx[h] = gidx[h] + 1
                sidx[h] = sidx[h] + GR
            return (*accs, *gidx, *sidx, jv + 1)

        def step_full(carry):
            accs, gidx, sidx, jv = unpack(carry)
            for h in range(NCH):
                plsc.store_scatter(ob, [zvec, sidx[h]], accs[h])
                v = plsc.load_gather(xb, [zvec, gidx[h]])
                accs[h] = accs[h] + v
                gidx[h] = gidx[h] + 1
                sidx[h] = sidx[h] + GR
            return (*accs, *gidx, *sidx, jv + 1)

        gidx0 = [PC * rh - iota for rh in rows_h]
        sidx0 = [rh - GR * iota for rh in rows_h]
        carry = (*accs, *gidx0, *sidx0, -iota)
        carry = plsc.parallel_loop(0, L - 1, unroll=5, carry=carry)(
            lambda t, cr: step_masked(cr))
        carry = plsc.parallel_loop(L - 1, PC - 1, unroll=7, carry=carry)(
            lambda t, cr: step_full(cr))
        carry = plsc.parallel_loop(PC - 1, PC + L - 1, unroll=5, carry=carry)(
            lambda t, cr: step_masked(cr))

        for h in range(NCH):
            tot_buf[pl.ds(g * GR + L * h, L)] = carry[h]

    def do_unit(k, u, xb, ob, sem_x, sem_o, other_xb, other_sem_x):
        _, _, gr0, c0 = unit_slices(u)

        @pl.when(u + 1 < nunits)
        def _():
            issue_x(u + 1, other_xb, other_sem_x)

        pltpu.make_async_copy(
            x_hbm.at[pl.ds(gr0, GR), pl.ds(c0, PC)], xb, sem_x).wait()

        @pl.when(k >= 1)
        def _():
            pltpu.make_async_copy(
                ob, out_hbm.at[pl.ds(c0, PC), pl.ds(gr0, GR)], sem_o).wait()

        sweep(u, xb, ob)
        pltpu.async_copy(ob, out_hbm.at[pl.ds(c0, PC), pl.ds(gr0, GR)], sem_o)

    issue_x(0, xb0, sem_x0)

    @pl.loop(0, npairs)
    def _(k):
        do_unit(k, 2 * k, xb0, ob0, sem_x0, sem_o0, xb1, sem_x1)
        do_unit(k, 2 * k + 1, xb1, ob1, sem_x1, sem_o1, xb0, sem_x0)

    pltpu.make_async_copy(ob0, out_hbm.at[pl.ds(0, PC), pl.ds(base, GR)],
                          sem_o0).wait()
    pltpu.make_async_copy(ob1, out_hbm.at[pl.ds(0, PC), pl.ds(base, GR)],
                          sem_o1).wait()

    @pl.when(wid != NUM_WORKERS - 1)
    def _():
        pltpu.sync_copy(tot_buf, tot_hbm.at[pl.ds(base, ROWS_PER_WORKER)])

    @pl.when(wid == NUM_WORKERS - 1)
    def _():
        pltpu.sync_copy(tot_buf.at[pl.ds(0, ROWS_PER_WORKER - GR)],
                        tot_hbm.at[pl.ds(base, ROWS_PER_WORKER - GR)])


def _patch_kernel(tot_ref, tail_ref, out_alias, out_hbm, sem):
    del out_alias
    cp1 = pltpu.make_async_copy(
        tot_ref, out_hbm.at[pl.ds(COLS, 1), pl.ds(0, ROWS_SC)], sem)
    cp1.start()
    cp1.wait()
    cp2 = pltpu.make_async_copy(
        tail_ref, out_hbm.at[:, pl.ds(ROWS_SC, TAIL)], sem)
    cp2.start()
    cp2.wait()


def kernel(x):
    mesh = plsc.VectorSubcoreMesh(core_axis_name="c", subcore_axis_name="s")
    cp = pltpu.CompilerParams()
    if "needs_layout_passes" in pltpu.CompilerParams.__dataclass_fields__:
        cp = dataclasses.replace(cp, needs_layout_passes=False)
    run = pl.kernel(
        _sc_kernel,
        out_type=(
            jax.ShapeDtypeStruct((COLS_OUT, ROWS_OUT), jnp.float32),
            jax.ShapeDtypeStruct((ROWS_OUT,), jnp.float32),
        ),
        mesh=mesh,
        compiler_params=cp,
        scratch_types=[
            pltpu.VMEM((GR, PC), jnp.float32),
            pltpu.VMEM((GR, PC), jnp.float32),
            pltpu.VMEM((PC, GR), jnp.float32),
            pltpu.VMEM((PC, GR), jnp.float32),
            pltpu.VMEM((ROWS_PER_WORKER,), jnp.float32),
            pltpu.SemaphoreType.DMA,
            pltpu.SemaphoreType.DMA,
            pltpu.SemaphoreType.DMA,
            pltpu.SemaphoreType.DMA,
        ],
    )
    out_t, totals = run(x)

    tail_x = lax.slice(x, (ROWS_SC, 0), (ROWS_OUT, COLS))
    tail_t = jnp.concatenate(
        [jnp.zeros((TAIL, 1), jnp.float32), jnp.cumsum(tail_x, axis=1)],
        axis=1).T
    tot_main = lax.slice(totals, (0,), (ROWS_SC,)).reshape(1, ROWS_SC)
    patch = pl.pallas_call(
        _patch_kernel,
        out_shape=jax.ShapeDtypeStruct((COLS_OUT, ROWS_OUT), jnp.float32),
        in_specs=[pl.BlockSpec(memory_space=pltpu.VMEM),
                  pl.BlockSpec(memory_space=pltpu.VMEM),
                  pl.BlockSpec(memory_space=pl.ANY)],
        out_specs=pl.BlockSpec(memory_space=pl.ANY),
        scratch_shapes=[pltpu.SemaphoreType.DMA],
        input_output_aliases={2: 0},
    )
    return patch(tot_main, tail_t, out_t).T


# speedup vs baseline: 1.0027x; 1.0027x over previous
import dataclasses

import jax
import jax.numpy as jnp
from jax import lax
from jax.experimental import pallas as pl
from jax.experimental.pallas import tpu as pltpu
from jax.experimental.pallas import tpu_sc as plsc

ROWS_IN = 65536
COLS = 1024
ROWS_OUT = ROWS_IN - 1
COLS_OUT = COLS + 1

NUM_WORKERS = 32
L = 16
GR = 128
NCH = GR // L
PC = 128
NPANEL = COLS // PC
ROWS_PER_WORKER = 2048
GROUPS_FULL = ROWS_PER_WORKER // GR
ROWS_SC = 65408
TAIL = ROWS_OUT - ROWS_SC


def _sc_kernel(x_hbm, out_hbm, tot_hbm,
               xb0, xb1, ob0, ob1, tot_buf,
               sem_x0, sem_x1, sem_o0, sem_o1):
    c = lax.axis_index("c")
    s = lax.axis_index("s")
    wid = s * 2 + c
    base = wid * ROWS_PER_WORKER

    iota = jnp.arange(L, dtype=jnp.int32)
    rows_h = [iota + L * h for h in range(NCH)]
    zeros_f = jnp.zeros((L,), jnp.float32)

    ngroups = jnp.where(wid == NUM_WORKERS - 1,
                        GROUPS_FULL - 1, GROUPS_FULL)
    nunits = ngroups * NPANEL
    npairs = nunits // 2

    def unit_slices(u):
        g = u // NPANEL
        p = lax.rem(u, NPANEL)
        return g, p, base + g * GR, p * PC

    def issue_x(u, xb, sem):
        _, _, gr0, c0 = unit_slices(u)
        pltpu.async_copy(x_hbm.at[pl.ds(gr0, GR), pl.ds(c0, PC)], xb, sem)

    def sweep(u, xb, ob):
        g, p, gr0, c0 = unit_slices(u)
        keep = jnp.where(p == 0, 0.0, 1.0)
        accs = [tot_buf[pl.ds(g * GR + L * h, L)] * keep for h in range(NCH)]

        zvec = jnp.zeros((L,), jnp.int32)

        def unpack(carry):
            accs = list(carry[0:NCH])
            gidx = list(carry[NCH:2 * NCH])
            sidx = list(carry[2 * NCH:3 * NCH])
            return accs, gidx, sidx, carry[-1]

        def step_masked(carry):
            accs, gidx, sidx, jv = unpack(carry)
            m = (jv >= 0) & (jv < PC)
            for h in range(NCH):
                plsc.store_scatter(ob, [zvec, sidx[h]], accs[h], mask=m)
                v = plsc.load_gather(xb, [zvec, gidx[h]], mask=m)
                accs[h] = accs[h] + jnp.where(m, v, 0.0)
                gidx[h] = gidx[h] + 1
                sidx[h] = sidx[h] + GR
            return (*accs, *gidx, *sidx, jv + 1)

        def step_full(carry):
            accs, gidx, sidx, jv = unpack(carry)
            for h in range(NCH):
                plsc.store_scatter(ob, [zvec, sidx[h]], accs[h])
                v = plsc.load_gather(xb, [zvec, gidx[h]])
                accs[h] = accs[h] + v
                gidx[h] = gidx[h] + 1
                sidx[h] = sidx[h] + GR
            return (*accs, *gidx, *sidx, jv + 1)

        gidx0 = [PC * rh - iota for rh in rows_h]
        sidx0 = [rh - GR * iota for rh in rows_h]
        carry = (*accs, *gidx0, *sidx0, -iota)
        carry = plsc.parallel_loop(0, L - 1, unroll=5, carry=carry)(
            lambda t, cr: step_masked(cr))
        carry = plsc.parallel_loop(L - 1, PC - 1, unroll=4, carry=carry)(
            lambda t, cr: step_full(cr))
        carry = plsc.parallel_loop(PC - 1, PC + L - 1, unroll=5, carry=carry)(
            lambda t, cr: step_masked(cr))

        for h in range(NCH):
            tot_buf[pl.ds(g * GR + L * h, L)] = carry[h]

    def do_unit(k, u, xb, ob, sem_x, sem_o, other_xb, other_sem_x):
        _, _, gr0, c0 = unit_slices(u)

        @pl.when(u + 1 < nunits)
        def _():
            issue_x(u + 1, other_xb, other_sem_x)

        pltpu.make_async_copy(
            x_hbm.at[pl.ds(gr0, GR), pl.ds(c0, PC)], xb, sem_x).wait()

        @pl.when(k >= 1)
        def _():
            pltpu.make_async_copy(
                ob, out_hbm.at[pl.ds(c0, PC), pl.ds(gr0, GR)], sem_o).wait()

        sweep(u, xb, ob)
        pltpu.async_copy(ob, out_hbm.at[pl.ds(c0, PC), pl.ds(gr0, GR)], sem_o)

    issue_x(0, xb0, sem_x0)

    @pl.loop(0, npairs)
    def _(k):
        do_unit(k, 2 * k, xb0, ob0, sem_x0, sem_o0, xb1, sem_x1)
        do_unit(k, 2 * k + 1, xb1, ob1, sem_x1, sem_o1, xb0, sem_x0)

    pltpu.make_async_copy(ob0, out_hbm.at[pl.ds(0, PC), pl.ds(base, GR)],
                          sem_o0).wait()
    pltpu.make_async_copy(ob1, out_hbm.at[pl.ds(0, PC), pl.ds(base, GR)],
                          sem_o1).wait()

    @pl.when(wid != NUM_WORKERS - 1)
    def _():
        pltpu.sync_copy(tot_buf, tot_hbm.at[pl.ds(base, ROWS_PER_WORKER)])

    @pl.when(wid == NUM_WORKERS - 1)
    def _():
        pltpu.sync_copy(tot_buf.at[pl.ds(0, ROWS_PER_WORKER - GR)],
                        tot_hbm.at[pl.ds(base, ROWS_PER_WORKER - GR)])


def _patch_kernel(tot_ref, tail_ref, out_alias, out_hbm, sem):
    del out_alias
    cp1 = pltpu.make_async_copy(
        tot_ref, out_hbm.at[pl.ds(COLS, 1), pl.ds(0, ROWS_SC)], sem)
    cp1.start()
    cp1.wait()
    cp2 = pltpu.make_async_copy(
        tail_ref, out_hbm.at[:, pl.ds(ROWS_SC, TAIL)], sem)
    cp2.start()
    cp2.wait()


def kernel(x):
    mesh = plsc.VectorSubcoreMesh(core_axis_name="c", subcore_axis_name="s")
    cp = pltpu.CompilerParams()
    if "needs_layout_passes" in pltpu.CompilerParams.__dataclass_fields__:
        cp = dataclasses.replace(cp, needs_layout_passes=False)
    run = pl.kernel(
        _sc_kernel,
        out_type=(
            jax.ShapeDtypeStruct((COLS_OUT, ROWS_OUT), jnp.float32),
            jax.ShapeDtypeStruct((ROWS_OUT,), jnp.float32),
        ),
        mesh=mesh,
        compiler_params=cp,
        scratch_types=[
            pltpu.VMEM((GR, PC), jnp.float32),
            pltpu.VMEM((GR, PC), jnp.float32),
            pltpu.VMEM((PC, GR), jnp.float32),
            pltpu.VMEM((PC, GR), jnp.float32),
            pltpu.VMEM((ROWS_PER_WORKER,), jnp.float32),
            pltpu.SemaphoreType.DMA,
            pltpu.SemaphoreType.DMA,
            pltpu.SemaphoreType.DMA,
            pltpu.SemaphoreType.DMA,
        ],
    )
    out_t, totals = run(x)

    tail_x = lax.slice(x, (ROWS_SC, 0), (ROWS_OUT, COLS))
    tail_t = jnp.concatenate(
        [jnp.zeros((TAIL, 1), jnp.float32), jnp.cumsum(tail_x, axis=1)],
        axis=1).T
    tot_main = lax.slice(totals, (0,), (ROWS_SC,)).reshape(1, ROWS_SC)
    patch = pl.pallas_call(
        _patch_kernel,
        out_shape=jax.ShapeDtypeStruct((COLS_OUT, ROWS_OUT), jnp.float32),
        in_specs=[pl.BlockSpec(memory_space=pltpu.VMEM),
                  pl.BlockSpec(memory_space=pltpu.VMEM),
                  pl.BlockSpec(memory_space=pl.ANY)],
        out_specs=pl.BlockSpec(memory_space=pl.ANY),
        scratch_shapes=[pltpu.SemaphoreType.DMA],
        input_output_aliases={2: 0},
    )
    return patch(tot_main, tail_t, out_t).T
